# Initial kernel scaffold; baseline (speedup 1.0000x reference)
import jax
import jax.numpy as jnp
from jax.experimental import pallas as pl


def kernel(src, tgt, src_table, tgt_table):
    raise NotImplementedError("write your pallas kernel here")


# speedup vs baseline: 4.8743x; 4.8743x over previous
import functools

import jax
import jax.numpy as jnp
from jax import lax
from jax.experimental import pallas as pl
from jax.experimental.pallas import tpu as pltpu
from jax.experimental.pallas import tpu_sc as plsc

EMB = 64
BATCH = 4096
SEQ = 50
N_TOK = BATCH * SEQ
NW = 32
PER_W = N_TOK // NW
SUB = 128
N_SUB = 10
CHUNK = SUB * N_SUB
N_CHUNK = PER_W // CHUNK


def _gather_one_table(table_hbm, idx_hbm, out_hbm, idx_v, rows_v, sem,
                      tok_base, out_base):
    for k in range(N_CHUNK):
        pltpu.sync_copy(idx_hbm.at[pl.ds(tok_base + k * CHUNK, CHUNK)], idx_v)
        copies = []
        for j in range(N_SUB):
            copies.append(pltpu.async_copy(
                table_hbm.at[idx_v.at[pl.ds(j * SUB, SUB)]],
                rows_v.at[pl.ds(j * SUB, SUB)],
                sem))
        for c in copies:
            c.wait()
        pltpu.sync_copy(rows_v,
                        out_hbm.at[pl.ds(out_base + k * CHUNK, CHUNK)])


@functools.partial(
    pl.kernel,
    mesh=plsc.VectorSubcoreMesh(core_axis_name="c", subcore_axis_name="s"),
    out_type=jax.ShapeDtypeStruct((2 * N_TOK, EMB), jnp.float32),
    scratch_types=[
        pltpu.VMEM((CHUNK,), jnp.int32),
        pltpu.VMEM((CHUNK, EMB), jnp.float32),
        pltpu.SemaphoreType.DMA,
    ],
    compiler_params=pltpu.CompilerParams(use_tc_tiling_on_sc=False),
)
def _emb_lookup(src_idx, tgt_idx, src_table, tgt_table, out_hbm,
                idx_v, rows_v, sem):
    info = plsc.get_sparse_core_info()
    wid = lax.axis_index("s") * info.num_cores + lax.axis_index("c")
    tok_base = wid * PER_W
    _gather_one_table(src_table, src_idx, out_hbm, idx_v, rows_v, sem,
                      tok_base, tok_base)
    _gather_one_table(tgt_table, tgt_idx, out_hbm, idx_v, rows_v, sem,
                      tok_base, N_TOK + tok_base)


def kernel(src, tgt, src_table, tgt_table):
    src_idx = src.reshape(N_TOK)
    tgt_idx = tgt.reshape(N_TOK)
    out = _emb_lookup(src_idx, tgt_idx, src_table, tgt_table)
    return out.reshape(2, BATCH, SEQ, EMB)
